# baseline (device time: 63669 ns/iter reference)
import jax
import jax.numpy as jnp
from jax import lax
from jax.experimental import pallas as pl
from jax.experimental.pallas import tpu as pltpu

B, S_SHARD, H, D = 2, 256, 8, 64
S_FULL = 2 * S_SHARD
SCALE = D ** -0.5


def kernel(Q, K, V):
    def body(q_ref, k_ref, v_ref, out_ref, kf_ref, vf_ref, send_sems, recv_sems):
        my_x = lax.axis_index("x")
        my_y = lax.axis_index("y")
        other_x = 1 - my_x

        barrier_sem = pltpu.get_barrier_semaphore()
        pl.semaphore_signal(
            barrier_sem, inc=1,
            device_id=(other_x, my_y), device_id_type=pltpu.DeviceIdType.MESH,
        )
        pl.semaphore_wait(barrier_sem, 1)

        kf_ref[:, pl.ds(my_x * S_SHARD, S_SHARD)] = k_ref[...]
        vf_ref[:, pl.ds(my_x * S_SHARD, S_SHARD)] = v_ref[...]

        rdma_k = pltpu.make_async_remote_copy(
            src_ref=k_ref,
            dst_ref=kf_ref.at[:, pl.ds(my_x * S_SHARD, S_SHARD)],
            send_sem=send_sems.at[0],
            recv_sem=recv_sems.at[0],
            device_id=(other_x, my_y),
            device_id_type=pltpu.DeviceIdType.MESH,
        )
        rdma_v = pltpu.make_async_remote_copy(
            src_ref=v_ref,
            dst_ref=vf_ref.at[:, pl.ds(my_x * S_SHARD, S_SHARD)],
            send_sem=send_sems.at[1],
            recv_sem=recv_sems.at[1],
            device_id=(other_x, my_y),
            device_id_type=pltpu.DeviceIdType.MESH,
        )
        rdma_k.start()
        rdma_v.start()
        rdma_k.wait()
        rdma_v.wait()

        for b in range(B):
            for h in range(H):
                q = q_ref[b, :, h, :]
                k = kf_ref[b, :, h, :]
                v = vf_ref[b, :, h, :]
                s = lax.dot_general(
                    q, k, (((1,), (1,)), ((), ())),
                    preferred_element_type=jnp.float32,
                ) * SCALE
                m = jnp.max(s, axis=-1, keepdims=True)
                p = jnp.exp(s - m)
                p = p / jnp.sum(p, axis=-1, keepdims=True)
                o = lax.dot_general(
                    p, v, (((1,), (0,)), ((), ())),
                    preferred_element_type=jnp.float32,
                )
                out_ref[b, :, h, :] = o

    return pl.pallas_call(
        body,
        out_shape=jax.ShapeDtypeStruct((B, S_SHARD, H, D), jnp.float32),
        in_specs=[
            pl.BlockSpec(memory_space=pltpu.VMEM),
            pl.BlockSpec(memory_space=pltpu.VMEM),
            pl.BlockSpec(memory_space=pltpu.VMEM),
        ],
        out_specs=pl.BlockSpec(memory_space=pltpu.VMEM),
        scratch_shapes=[
            pltpu.VMEM((B, S_FULL, H, D), jnp.float32),
            pltpu.VMEM((B, S_FULL, H, D), jnp.float32),
            pltpu.SemaphoreType.DMA((2,)),
            pltpu.SemaphoreType.DMA((2,)),
        ],
        compiler_params=pltpu.CompilerParams(collective_id=0),
    )(Q, K, V)


# device time: 57633 ns/iter; 1.1047x vs baseline; 1.1047x over previous
import jax
import jax.numpy as jnp
from jax import lax
from jax.experimental import pallas as pl
from jax.experimental.pallas import tpu as pltpu

B, S_SHARD, H, D = 2, 256, 8, 64
S_FULL = 2 * S_SHARD
SCALE = D ** -0.5


def kernel(Q, K, V):
    Qt = jnp.transpose(Q, (0, 2, 1, 3))
    Kt = jnp.transpose(K, (0, 2, 1, 3))
    Vt = jnp.transpose(V, (0, 2, 1, 3))

    def body(q_ref, k_ref, v_ref, out_ref, kf_ref, vf_ref, send_sems, recv_sems):
        my_x = lax.axis_index("x")
        my_y = lax.axis_index("y")
        other_x = 1 - my_x

        barrier_sem = pltpu.get_barrier_semaphore()
        pl.semaphore_signal(
            barrier_sem, inc=1,
            device_id=(other_x, my_y), device_id_type=pltpu.DeviceIdType.MESH,
        )
        pl.semaphore_wait(barrier_sem, 1)

        kf_ref[:, :, pl.ds(my_x * S_SHARD, S_SHARD)] = k_ref[...]
        vf_ref[:, :, pl.ds(my_x * S_SHARD, S_SHARD)] = v_ref[...]

        rdma_k = pltpu.make_async_remote_copy(
            src_ref=k_ref,
            dst_ref=kf_ref.at[:, :, pl.ds(my_x * S_SHARD, S_SHARD)],
            send_sem=send_sems.at[0],
            recv_sem=recv_sems.at[0],
            device_id=(other_x, my_y),
            device_id_type=pltpu.DeviceIdType.MESH,
        )
        rdma_v = pltpu.make_async_remote_copy(
            src_ref=v_ref,
            dst_ref=vf_ref.at[:, :, pl.ds(my_x * S_SHARD, S_SHARD)],
            send_sem=send_sems.at[1],
            recv_sem=recv_sems.at[1],
            device_id=(other_x, my_y),
            device_id_type=pltpu.DeviceIdType.MESH,
        )
        rdma_k.start()
        rdma_v.start()
        rdma_k.wait()
        rdma_v.wait()

        for b in range(B):
            for h in range(H):
                q = q_ref[b, h]
                k = kf_ref[b, h]
                v = vf_ref[b, h]
                s = lax.dot_general(
                    q, k, (((1,), (1,)), ((), ())),
                    preferred_element_type=jnp.float32,
                ) * SCALE
                m = jnp.max(s, axis=-1, keepdims=True)
                p = jnp.exp(s - m)
                p = p / jnp.sum(p, axis=-1, keepdims=True)
                o = lax.dot_general(
                    p, v, (((1,), (0,)), ((), ())),
                    preferred_element_type=jnp.float32,
                )
                out_ref[b, h] = o

    out_t = pl.pallas_call(
        body,
        out_shape=jax.ShapeDtypeStruct((B, H, S_SHARD, D), jnp.float32),
        in_specs=[
            pl.BlockSpec(memory_space=pltpu.VMEM),
            pl.BlockSpec(memory_space=pltpu.VMEM),
            pl.BlockSpec(memory_space=pltpu.VMEM),
        ],
        out_specs=pl.BlockSpec(memory_space=pltpu.VMEM),
        scratch_shapes=[
            pltpu.VMEM((B, H, S_FULL, D), jnp.float32),
            pltpu.VMEM((B, H, S_FULL, D), jnp.float32),
            pltpu.SemaphoreType.DMA((2,)),
            pltpu.SemaphoreType.DMA((2,)),
        ],
        compiler_params=pltpu.CompilerParams(collective_id=0),
    )(Qt, Kt, Vt)
    return jnp.transpose(out_t, (0, 2, 1, 3))


# device time: 24360 ns/iter; 2.6137x vs baseline; 2.3659x over previous
import jax
import jax.numpy as jnp
from jax import lax
from jax.experimental import pallas as pl
from jax.experimental.pallas import tpu as pltpu

B, S_SHARD, H, D = 2, 256, 8, 64
BH = B * H
HALF = BH // 2
SCALE = D ** -0.5


def kernel(Q, K, V):
    Qb = jnp.transpose(Q.astype(jnp.bfloat16), (0, 2, 1, 3)).reshape(BH, S_SHARD, D)
    Kb = jnp.transpose(K.astype(jnp.bfloat16), (0, 2, 1, 3)).reshape(BH, S_SHARD, D)
    Vb = jnp.transpose(V.astype(jnp.bfloat16), (0, 2, 1, 3)).reshape(BH, S_SHARD, D)

    def body(q_ref, k_ref, v_ref, out_ref, kr_ref, vr_ref,
             x_send, x_recv, y_send, y_recv):
        my_x = lax.axis_index("x")
        my_y = lax.axis_index("y")
        other_x = 1 - my_x
        other_y = 1 - my_y
        base = my_y * HALF
        obase = other_y * HALF

        barrier_sem = pltpu.get_barrier_semaphore()
        for nbr in ((other_x, my_y), (my_x, other_y)):
            pl.semaphore_signal(
                barrier_sem, inc=1,
                device_id=nbr, device_id_type=pltpu.DeviceIdType.MESH,
            )
        pl.semaphore_wait(barrier_sem, 2)

        x_rdmas = []
        for j in range(HALF):
            row = pl.ds(base + j, 1)
            rk = pltpu.make_async_remote_copy(
                src_ref=k_ref.at[row], dst_ref=kr_ref.at[row],
                send_sem=x_send.at[j], recv_sem=x_recv.at[j],
                device_id=(other_x, my_y),
                device_id_type=pltpu.DeviceIdType.MESH,
            )
            rv = pltpu.make_async_remote_copy(
                src_ref=v_ref.at[row], dst_ref=vr_ref.at[row],
                send_sem=x_send.at[HALF + j], recv_sem=x_recv.at[HALF + j],
                device_id=(other_x, my_y),
                device_id_type=pltpu.DeviceIdType.MESH,
            )
            rk.start()
            rv.start()
            x_rdmas.append((rk, rv))

        y_rdmas = []
        for j in range(HALF):
            i = base + j
            rk, rv = x_rdmas[j]
            rk.wait_recv()
            rv.wait_recv()
            q = q_ref[i]
            s1 = lax.dot_general(
                q, k_ref[i], (((1,), (1,)), ((), ())),
                preferred_element_type=jnp.float32) * SCALE
            s2 = lax.dot_general(
                q, kr_ref[i], (((1,), (1,)), ((), ())),
                preferred_element_type=jnp.float32) * SCALE
            m = jnp.maximum(jnp.max(s1, axis=-1, keepdims=True),
                            jnp.max(s2, axis=-1, keepdims=True))
            p1 = jnp.exp(s1 - m)
            p2 = jnp.exp(s2 - m)
            l = (jnp.sum(p1, axis=-1, keepdims=True)
                 + jnp.sum(p2, axis=-1, keepdims=True))
            p1b = p1.astype(jnp.bfloat16)
            p2b = p2.astype(jnp.bfloat16)
            o = (lax.dot_general(p1b, v_ref[i], (((1,), (0,)), ((), ())),
                                 preferred_element_type=jnp.float32)
                 + lax.dot_general(p2b, vr_ref[i], (((1,), (0,)), ((), ())),
                                   preferred_element_type=jnp.float32))
            out_ref[i] = o / l

            row = pl.ds(i, 1)
            fo = pltpu.make_async_remote_copy(
                src_ref=out_ref.at[row], dst_ref=out_ref.at[row],
                send_sem=y_send.at[j], recv_sem=y_recv.at[j],
                device_id=(my_x, other_y),
                device_id_type=pltpu.DeviceIdType.MESH,
            )
            fo.start()
            y_rdmas.append(fo)

        for j in range(HALF):
            row = pl.ds(obase + j, 1)
            pltpu.make_async_remote_copy(
                src_ref=out_ref.at[row], dst_ref=out_ref.at[row],
                send_sem=y_send.at[j], recv_sem=y_recv.at[j],
                device_id=(my_x, other_y),
                device_id_type=pltpu.DeviceIdType.MESH,
            ).wait_recv()

        for rk, rv in x_rdmas:
            rk.wait_send()
            rv.wait_send()
        for fo in y_rdmas:
            fo.wait_send()

    out_t = pl.pallas_call(
        body,
        out_shape=jax.ShapeDtypeStruct((BH, S_SHARD, D), jnp.float32),
        in_specs=[pl.BlockSpec(memory_space=pltpu.VMEM)] * 3,
        out_specs=pl.BlockSpec(memory_space=pltpu.VMEM),
        scratch_shapes=[
            pltpu.VMEM((BH, S_SHARD, D), jnp.bfloat16),
            pltpu.VMEM((BH, S_SHARD, D), jnp.bfloat16),
            pltpu.SemaphoreType.DMA((2 * HALF,)),
            pltpu.SemaphoreType.DMA((2 * HALF,)),
            pltpu.SemaphoreType.DMA((HALF,)),
            pltpu.SemaphoreType.DMA((HALF,)),
        ],
        compiler_params=pltpu.CompilerParams(collective_id=0),
    )(Qb, Kb, Vb)
    return jnp.transpose(out_t.reshape(B, H, S_SHARD, D), (0, 2, 1, 3))
